# Initial kernel scaffold; baseline (speedup 1.0000x reference)
import jax
import jax.numpy as jnp
from jax.experimental import pallas as pl


def kernel(x, edge_index, edge_weights, Ws, bs):
    raise NotImplementedError("write your pallas kernel here")


# speedup vs baseline: 4.0011x; 4.0011x over previous
import functools

import jax
import jax.numpy as jnp
from jax import lax
from jax.experimental import pallas as pl
from jax.experimental.pallas import tpu as pltpu
from jax.experimental.pallas import tpu_sc as plsc

NC, NS, LANES = 2, 16, 16
NW = NC * NS


def _sc_aggregate(N, D, E):
    EPW = E // NW
    CH = 80
    NCHUNK = EPW // CH
    RPT = (N // NS) & ~7
    TAIL = N - RPT * NS
    ZR = RPT // 6
    NZ = RPT // ZR
    NV = D // LANES

    mesh = plsc.VectorSubcoreMesh(core_axis_name="c", subcore_axis_name="s")

    @functools.partial(
        pl.kernel,
        mesh=mesh,
        out_type=jax.ShapeDtypeStruct((NC, N, D), jnp.float32),
        scratch_types=[
            pltpu.VMEM((CH,), jnp.int32),
            pltpu.VMEM((CH,), jnp.int32),
            pltpu.VMEM((CH,), jnp.float32),
            pltpu.VMEM((CH, D), jnp.float32),
            pltpu.VMEM((ZR, D), jnp.float32),
            pltpu.VMEM_SHARED((N, D), jnp.float32),
            pltpu.SemaphoreType.DMA,
        ],
    )
    def body(h_hbm, src_hbm, dst_hbm, w_hbm, out_hbm,
             sidx, didx, wv, rows, zbuf, acc, sem):
        c = lax.axis_index("c")
        s = lax.axis_index("s")
        wid = c * NS + s

        def zrow(r, carry):
            for v in range(NV):
                zbuf[r, pl.ds(v * LANES, LANES)] = jnp.zeros((LANES,), jnp.float32)
            return carry
        lax.fori_loop(0, ZR, zrow, 0)
        for k in range(NZ):
            pltpu.sync_copy(zbuf, acc.at[pl.ds(s * RPT + k * ZR, ZR)])

        @pl.when(s == NS - 1)
        def _zero_tail():
            pltpu.sync_copy(zbuf.at[pl.ds(0, TAIL)],
                            acc.at[pl.ds(NS * RPT, TAIL)])
        plsc.subcore_barrier()

        base0 = wid * EPW

        def chunk(j, carry):
            base = base0 + j * CH
            pltpu.sync_copy(src_hbm.at[pl.ds(base, CH)], sidx)
            pltpu.sync_copy(dst_hbm.at[pl.ds(base, CH)], didx)
            pltpu.sync_copy(w_hbm.at[pl.ds(base, CH)], wv)
            pltpu.async_copy(h_hbm.at[sidx], rows, sem).wait()

            def egroup(g, ecarry):
                e0 = g * LANES
                w16 = wv[pl.ds(e0, LANES)]
                for el in range(LANES):
                    we = w16[el]
                    for v in range(NV):
                        sl = pl.ds(v * LANES, LANES)
                        rows[e0 + el, sl] = rows[e0 + el, sl] * we
                return ecarry
            lax.fori_loop(0, CH // LANES, egroup, 0)

            pltpu.sync_copy(rows, acc.at[didx], add=True)
            return carry
        lax.fori_loop(0, NCHUNK, chunk, 0)

        plsc.subcore_barrier()
        pltpu.sync_copy(acc.at[pl.ds(s * RPT, RPT)],
                        out_hbm.at[c, pl.ds(s * RPT, RPT)])

        @pl.when(s == NS - 1)
        def _write_tail():
            pltpu.sync_copy(acc.at[pl.ds(NS * RPT, TAIL)],
                            out_hbm.at[c, pl.ds(NS * RPT, TAIL)])

    return body


def _tc_update(N, D):
    R = 1000
    G = N // R

    def body(p_ref, h_ref, w1_ref, w2_ref, b_ref, o_ref):
        a = p_ref[0] + p_ref[1]
        nrm = jnp.sqrt(jnp.sum(a * a, axis=1, keepdims=True))
        na = a / jnp.maximum(nrm, 1e-12)
        out = (jnp.dot(na, w1_ref[...], preferred_element_type=jnp.float32)
               + jnp.dot(h_ref[...], w2_ref[...], preferred_element_type=jnp.float32)
               + b_ref[...])
        n2 = jnp.sqrt(jnp.sum(out * out, axis=1, keepdims=True))
        o_ref[...] = out / jnp.maximum(n2, 1e-12)

    return pl.pallas_call(
        body,
        grid=(G,),
        in_specs=[
            pl.BlockSpec((NC, R, D), lambda i: (0, i, 0)),
            pl.BlockSpec((R, D), lambda i: (i, 0)),
            pl.BlockSpec((D, D), lambda i: (0, 0)),
            pl.BlockSpec((D, D), lambda i: (0, 0)),
            pl.BlockSpec((1, D), lambda i: (0, 0)),
        ],
        out_specs=pl.BlockSpec((R, D), lambda i: (i, 0)),
        out_shape=jax.ShapeDtypeStruct((N, D), jnp.float32),
    )


def kernel(x, edge_index, edge_weights, Ws, bs):
    N, D = x.shape
    E = edge_weights.shape[0]
    L = Ws.shape[0]
    src = edge_index[0].astype(jnp.int32)
    dst = edge_index[1].astype(jnp.int32)
    w = edge_weights.astype(jnp.float32)

    sc = _sc_aggregate(N, D, E)
    tc = _tc_update(N, D)

    h = x
    for l in range(L):
        parts = sc(h, src, dst, w)
        h = tc(parts, h, Ws[l, :D], Ws[l, D:], bs[l].reshape(1, D))
    return h


# speedup vs baseline: 12.4726x; 3.1173x over previous
import functools

import jax
import jax.numpy as jnp
from jax import lax
from jax.experimental import pallas as pl
from jax.experimental.pallas import tpu as pltpu
from jax.experimental.pallas import tpu_sc as plsc

NC, NS, LANES = 2, 16, 16
NW = NC * NS


def _sc_aggregate(N, D, E):
    EPW = E // NW
    CH = 40
    NCHUNK = EPW // CH
    RPT = (N // NS) & ~7
    TAIL = N - RPT * NS
    ZR = 24
    NZ = RPT // ZR
    NV = D // LANES

    NB = 5
    GRP = NCHUNK // NB
    mesh = plsc.VectorSubcoreMesh(core_axis_name="c", subcore_axis_name="s")

    @functools.partial(
        pl.kernel,
        mesh=mesh,
        out_type=jax.ShapeDtypeStruct((NC, N, D), jnp.float32),
        scratch_types=[
            pltpu.VMEM((NB, CH), jnp.int32),
            pltpu.VMEM((NB, CH), jnp.int32),
            pltpu.VMEM((NB, CH), jnp.float32),
            pltpu.VMEM((NB, CH, D), jnp.float32),
            pltpu.VMEM((ZR, D), jnp.float32),
            pltpu.VMEM_SHARED((N, D), jnp.float32),
            pltpu.SemaphoreType.DMA((NB,)),
            pltpu.SemaphoreType.DMA((NB,)),
            pltpu.SemaphoreType.DMA((NB,)),
        ],
    )
    def body(h_hbm, src_hbm, dst_hbm, w_hbm, out_hbm,
             sidx, didx, wv, rows, zbuf, acc, semi, semg, sems):
        c = lax.axis_index("c")
        s = lax.axis_index("s")
        wid = c * NS + s

        def zrow(r, carry):
            for v in range(NV):
                zbuf[r, pl.ds(v * LANES, LANES)] = jnp.zeros((LANES,), jnp.float32)
            return carry
        lax.fori_loop(0, ZR, zrow, 0)
        for k in range(NZ):
            pltpu.sync_copy(zbuf, acc.at[pl.ds(s * RPT + k * ZR, ZR)])

        @pl.when(s == NS - 1)
        def _zero_tail():
            pltpu.sync_copy(zbuf.at[pl.ds(0, TAIL)],
                            acc.at[pl.ds(NS * RPT, TAIL)])
        plsc.subcore_barrier()

        base0 = wid * EPW

        def issue_idx(t, b):
            base = base0 + t * CH
            pltpu.async_copy(src_hbm.at[pl.ds(base, CH)], sidx.at[b], semi.at[b])
            pltpu.async_copy(dst_hbm.at[pl.ds(base, CH)], didx.at[b], semi.at[b])
            pltpu.async_copy(w_hbm.at[pl.ds(base, CH)], wv.at[b], semi.at[b])

        def wait_idx(b):
            pltpu.make_async_copy(src_hbm.at[pl.ds(base0, CH)], sidx.at[b], semi.at[b]).wait()
            pltpu.make_async_copy(dst_hbm.at[pl.ds(base0, CH)], didx.at[b], semi.at[b]).wait()
            pltpu.make_async_copy(w_hbm.at[pl.ds(base0, CH)], wv.at[b], semi.at[b]).wait()

        def issue_gather(b):
            pltpu.async_copy(h_hbm.at[sidx.at[b]], rows.at[b], semg.at[b])

        def wait_gather(b):
            pltpu.make_async_copy(h_hbm.at[sidx.at[b]], rows.at[b], semg.at[b]).wait()

        def issue_scatter(b):
            pltpu.async_copy(rows.at[b], acc.at[didx.at[b]], sems.at[b], add=True)

        def wait_scatter(b):
            pltpu.make_async_copy(rows.at[b], acc.at[didx.at[b]], sems.at[b]).wait()

        for t in range(3):
            issue_idx(t, t)
        for t in range(2):
            wait_idx(t)
            issue_gather(t)

        def group(g, carry):
            for b in range(NB):
                j = g * NB + b
                b3 = (b + 3) % NB
                t3 = j + 3

                @pl.when(t3 < NCHUNK)
                def _pf_idx():
                    @pl.when(j >= 2)
                    def _drain_scatter():
                        wait_scatter(b3)
                    issue_idx(t3, b3)

                b2 = (b + 2) % NB
                t2 = j + 2

                @pl.when(t2 < NCHUNK)
                def _pf_gather():
                    wait_idx(b2)
                    issue_gather(b2)

                wait_gather(b)

                def egroup(gg, ecarry):
                    e0 = gg * LANES
                    w16 = wv[b, pl.ds(e0, LANES)]
                    for el in range(LANES):
                        we = w16[el]
                        for v in range(NV):
                            sl = pl.ds(v * LANES, LANES)
                            rows[b, e0 + el, sl] = rows[b, e0 + el, sl] * we
                    return ecarry
                lax.fori_loop(0, CH // LANES, egroup, 0)

                issue_scatter(b)
            return carry
        lax.fori_loop(0, GRP, group, 0)

        for b in range(NB):
            wait_scatter(b)

        plsc.subcore_barrier()
        pltpu.sync_copy(acc.at[pl.ds(s * RPT, RPT)],
                        out_hbm.at[c, pl.ds(s * RPT, RPT)])

        @pl.when(s == NS - 1)
        def _write_tail():
            pltpu.sync_copy(acc.at[pl.ds(NS * RPT, TAIL)],
                            out_hbm.at[c, pl.ds(NS * RPT, TAIL)])

    return body


def _tc_update(N, D):
    R = 1000
    G = N // R

    def body(p_ref, h_ref, w1_ref, w2_ref, b_ref, o_ref):
        a = p_ref[0] + p_ref[1]
        nrm = jnp.sqrt(jnp.sum(a * a, axis=1, keepdims=True))
        na = a / jnp.maximum(nrm, 1e-12)
        out = (jnp.dot(na, w1_ref[...], preferred_element_type=jnp.float32)
               + jnp.dot(h_ref[...], w2_ref[...], preferred_element_type=jnp.float32)
               + b_ref[...])
        n2 = jnp.sqrt(jnp.sum(out * out, axis=1, keepdims=True))
        o_ref[...] = out / jnp.maximum(n2, 1e-12)

    return pl.pallas_call(
        body,
        grid=(G,),
        in_specs=[
            pl.BlockSpec((NC, R, D), lambda i: (0, i, 0)),
            pl.BlockSpec((R, D), lambda i: (i, 0)),
            pl.BlockSpec((D, D), lambda i: (0, 0)),
            pl.BlockSpec((D, D), lambda i: (0, 0)),
            pl.BlockSpec((1, D), lambda i: (0, 0)),
        ],
        out_specs=pl.BlockSpec((R, D), lambda i: (i, 0)),
        out_shape=jax.ShapeDtypeStruct((N, D), jnp.float32),
    )


def kernel(x, edge_index, edge_weights, Ws, bs):
    N, D = x.shape
    E = edge_weights.shape[0]
    L = Ws.shape[0]
    src = edge_index[0].astype(jnp.int32)
    dst = edge_index[1].astype(jnp.int32)
    w = edge_weights.astype(jnp.float32)

    sc = _sc_aggregate(N, D, E)
    tc = _tc_update(N, D)

    h = x
    for l in range(L):
        parts = sc(h, src, dst, w)
        h = tc(parts, h, Ws[l, :D], Ws[l, D:], bs[l].reshape(1, D))
    return h
